# baseline (speedup 1.0000x reference)
import functools
import jax
import jax.numpy as jnp
from jax.experimental import pallas as pl
from jax.experimental.pallas import tpu as pltpu

DIM = 768
N = 2048
RANK = 64
TOPK = 16
BLK = 256
CBLK = 256
CHUNKS = BLK // CBLK
NT = N // BLK
F32 = jnp.float32
NEG = -3.4e38


def _xln(x, g, b):
    m = x.mean(-1, keepdims=True)
    v = x.var(-1, keepdims=True)
    return (x - m) / jnp.sqrt(v + 1e-5) * g + b


def _softplus(x):
    return jnp.maximum(x, 0.0) + jnp.log1p(jnp.exp(-jnp.abs(x)))


def _dirs(x):
    n = jnp.sqrt(jnp.sum(x * x, axis=-1, keepdims=True)) + 1e-6
    return x / n


def _st4(x):
    return x.reshape(x.shape[0], NT, 1, BLK)


_SPEC_ST = pl.BlockSpec((None, None, 1, BLK), lambda i, t: (i, t, 0, 0))
_SPEC_VAL = pl.BlockSpec((None, BLK, DIM), lambda i, t: (i, t, 0))
_SPEC_QK = pl.BlockSpec((None, BLK, RANK), lambda i, t: (i, t, 0))
_SPEC_W = pl.BlockSpec((DIM, RANK), lambda i, t: (0, 0))


def _kstate_body(s_ref, o_ref):
    s = s_ref[...]
    a = jnp.abs(s)
    mx = jnp.max(a, axis=-1, keepdims=True)
    e = jnp.exp(a - mx)
    o_ref[...] = jnp.sign(s) * e / jnp.sum(e, axis=-1, keepdims=True)


def _kstate(init_state):
    return pl.pallas_call(
        _kstate_body,
        out_shape=jax.ShapeDtypeStruct((1, N), F32),
    )(init_state.reshape(1, N))


def _prep_body(qsrc_ref, ksrc_ref, st_ref, wq_ref, wk_ref,
               q_ref, k_ref, dirs_ref, gate_ref):
    qs = qsrc_ref[...]
    q_ref[...] = jnp.dot(qs, wq_ref[...], preferred_element_type=F32)
    k_ref[...] = jnp.dot(ksrc_ref[...], wk_ref[...], preferred_element_type=F32)
    dirs_ref[...] = _dirs(qs)
    gate_ref[...] = _softplus(st_ref[...])


def _prep(qsrc, ksrc, state, wq, wk):
    B = qsrc.shape[0]
    return pl.pallas_call(
        _prep_body,
        grid=(B, NT),
        in_specs=[_SPEC_VAL, _SPEC_VAL, _SPEC_ST, _SPEC_W, _SPEC_W],
        out_specs=[_SPEC_QK, _SPEC_QK, _SPEC_VAL, _SPEC_ST],
        out_shape=[
            jax.ShapeDtypeStruct((B, N, RANK), F32),
            jax.ShapeDtypeStruct((B, N, RANK), F32),
            jax.ShapeDtypeStruct((B, N, DIM), F32),
            jax.ShapeDtypeStruct((B, NT, 1, BLK), F32),
        ],
    )(qsrc, ksrc, _st4(state), wq, wk)


def _route_body(kd_ref, q_ref, gate_ref, dirs_ref, so_ref, vo_ref):
    q = q_ref[...]
    gate = gate_ref[...]
    dirs = dirs_ref[...]
    for c in range(CHUNKS):
        kd = kd_ref[pl.ds(c * CBLK, CBLK), :]
        s = jax.lax.dot_general(kd, q, (((1,), (1,)), ((), ())),
                                preferred_element_type=F32) * 0.125
        a = jnp.abs(s)
        m = jnp.max(a, axis=-1, keepdims=True)
        thr = m
        for _ in range(TOPK - 1):
            cur = jnp.where(a >= thr, NEG, a)
            thr = jnp.max(cur, axis=-1, keepdims=True)
        e = jnp.where(a >= thr, jnp.exp(a - m), 0.0)
        z = jnp.sum(e, axis=-1, keepdims=True)
        w = jnp.sign(s) * e * gate / z
        so = jnp.sum(w, axis=-1)
        dv = jnp.dot(w, dirs, preferred_element_type=F32)
        so_ref[0, pl.ds(c * CBLK, CBLK)] = so
        vo_ref[pl.ds(c * CBLK, CBLK), :] = dv


def _route(kd, q, gate4, dirs):
    B = kd.shape[0]
    gate_row = gate4.reshape(B, 1, N)
    so4, vo = pl.pallas_call(
        _route_body,
        grid=(B, NT),
        in_specs=[
            _SPEC_QK,
            pl.BlockSpec((None, N, RANK), lambda i, t: (i, 0, 0)),
            pl.BlockSpec((None, 1, N), lambda i, t: (i, 0, 0)),
            pl.BlockSpec((None, N, DIM), lambda i, t: (i, 0, 0)),
        ],
        out_specs=[_SPEC_ST, _SPEC_VAL],
        out_shape=[
            jax.ShapeDtypeStruct((B, NT, 1, BLK), F32),
            jax.ShapeDtypeStruct((B, N, DIM), F32),
        ],
    )(kd, q, gate_row, dirs)
    return so4.reshape(B, N), vo


def kernel(b_state, b_val, init_state, init_val, bk_Wq, bk_Wk, kb_Wq, kb_Wk,
           pp_Wq, pp_Wk, kv_g, kv_b, br_g, br_b, pn_g, pn_b):
    B = b_state.shape[0]

    k_state1 = _kstate(init_state)
    kval = _xln(jnp.broadcast_to(init_val[None], (B, N, DIM)), kv_g, kv_b)
    nk = _xln(kval, kv_g, kv_b)

    q1, k1, dirs1, gate1 = _prep(b_val, nk, b_state, bk_Wq, bk_Wk)
    d_state1, d_val1 = _route(k1, q1, gate1, dirs1)
    routed_state = k_state1 + d_state1
    routed_val = _xln(kval + d_val1, kv_g, kv_b)

    nv = _xln(routed_val, pn_g, pn_b)
    q2, k2, dirs2, gate2 = _prep(nv, nv, routed_state, pp_Wq, pp_Wk)
    d_state2, d_val2 = _route(k2, q2, gate2, dirs2)
    prop_state = routed_state + d_state2
    prop_val = _xln(routed_val + d_val2, kv_g, kv_b)

    nk2 = _xln(prop_val, kv_g, kv_b)
    nb = _xln(b_val, br_g, br_b)
    q3, k3, dirs3, gate3 = _prep(nk2, nb, prop_state, kb_Wq, kb_Wk)
    bd_state, bd_val = _route(k3, q3, gate3, dirs3)

    return (routed_state, routed_val, prop_state, prop_val, bd_state, bd_val)


# speedup vs baseline: 1.0883x; 1.0883x over previous
import functools
import jax
import jax.numpy as jnp
from jax.experimental import pallas as pl
from jax.experimental.pallas import tpu as pltpu

DIM = 768
N = 2048
RANK = 64
TOPK = 16
BLK = 1024
CBLK = 256
CHUNKS = BLK // CBLK
NT = N // BLK
F32 = jnp.float32
NEG = -3.4e38


def _xln(x, g, b):
    m = x.mean(-1, keepdims=True)
    v = x.var(-1, keepdims=True)
    return (x - m) / jnp.sqrt(v + 1e-5) * g + b


def _softplus(x):
    return jnp.maximum(x, 0.0) + jnp.log1p(jnp.exp(-jnp.abs(x)))


def _dirs(x):
    n = jnp.sqrt(jnp.sum(x * x, axis=-1, keepdims=True)) + 1e-6
    return x / n


def _st4(x):
    return x.reshape(x.shape[0], NT, 1, BLK)


_SPEC_ST = pl.BlockSpec((None, None, 1, BLK), lambda i, t: (i, t, 0, 0))
_SPEC_VAL = pl.BlockSpec((None, BLK, DIM), lambda i, t: (i, t, 0))
_SPEC_QK = pl.BlockSpec((None, BLK, RANK), lambda i, t: (i, t, 0))
_SPEC_W = pl.BlockSpec((DIM, RANK), lambda i, t: (0, 0))


def _kstate_body(s_ref, o_ref):
    s = s_ref[...]
    a = jnp.abs(s)
    mx = jnp.max(a, axis=-1, keepdims=True)
    e = jnp.exp(a - mx)
    o_ref[...] = jnp.sign(s) * e / jnp.sum(e, axis=-1, keepdims=True)


def _kstate(init_state):
    return pl.pallas_call(
        _kstate_body,
        out_shape=jax.ShapeDtypeStruct((1, N), F32),
    )(init_state.reshape(1, N))


def _prep_body(qsrc_ref, ksrc_ref, st_ref, wq_ref, wk_ref,
               q_ref, k_ref, dirs_ref, gate_ref):
    qs = qsrc_ref[...]
    q_ref[...] = jnp.dot(qs, wq_ref[...], preferred_element_type=F32)
    k_ref[...] = jnp.dot(ksrc_ref[...], wk_ref[...], preferred_element_type=F32)
    dirs_ref[...] = _dirs(qs)
    gate_ref[...] = _softplus(st_ref[...])


def _prep(qsrc, ksrc, state, wq, wk):
    B = qsrc.shape[0]
    return pl.pallas_call(
        _prep_body,
        grid=(B, NT),
        in_specs=[_SPEC_VAL, _SPEC_VAL, _SPEC_ST, _SPEC_W, _SPEC_W],
        out_specs=[_SPEC_QK, _SPEC_QK, _SPEC_VAL, _SPEC_ST],
        out_shape=[
            jax.ShapeDtypeStruct((B, N, RANK), F32),
            jax.ShapeDtypeStruct((B, N, RANK), F32),
            jax.ShapeDtypeStruct((B, N, DIM), F32),
            jax.ShapeDtypeStruct((B, NT, 1, BLK), F32),
        ],
    )(qsrc, ksrc, _st4(state), wq, wk)


def _route_body(kd_ref, q_ref, gate_ref, dirs_ref, so_ref, vo_ref):
    q = q_ref[...]
    gate = gate_ref[...]
    dirs = dirs_ref[...]
    for c in range(CHUNKS):
        kd = kd_ref[pl.ds(c * CBLK, CBLK), :]
        s = jax.lax.dot_general(kd, q, (((1,), (1,)), ((), ())),
                                preferred_element_type=F32) * 0.125
        a = jnp.abs(s)
        m = jnp.max(a, axis=-1, keepdims=True)
        thr = m
        for _ in range(TOPK - 1):
            cur = jnp.where(a >= thr, NEG, a)
            thr = jnp.max(cur, axis=-1, keepdims=True)
        e = jnp.where(a >= thr, jnp.exp(a - m), 0.0)
        z = jnp.sum(e, axis=-1, keepdims=True)
        w = jnp.sign(s) * e * gate / z
        so = jnp.sum(w, axis=-1)
        dv = jnp.dot(w, dirs, preferred_element_type=F32)
        so_ref[0, pl.ds(c * CBLK, CBLK)] = so
        vo_ref[pl.ds(c * CBLK, CBLK), :] = dv


def _route(kd, q, gate4, dirs):
    B = kd.shape[0]
    gate_row = gate4.reshape(B, 1, N)
    so4, vo = pl.pallas_call(
        _route_body,
        grid=(B, NT),
        in_specs=[
            _SPEC_QK,
            pl.BlockSpec((None, N, RANK), lambda i, t: (i, 0, 0)),
            pl.BlockSpec((None, 1, N), lambda i, t: (i, 0, 0)),
            pl.BlockSpec((None, N, DIM), lambda i, t: (i, 0, 0)),
        ],
        out_specs=[_SPEC_ST, _SPEC_VAL],
        out_shape=[
            jax.ShapeDtypeStruct((B, NT, 1, BLK), F32),
            jax.ShapeDtypeStruct((B, N, DIM), F32),
        ],
    )(kd, q, gate_row, dirs)
    return so4.reshape(B, N), vo


def kernel(b_state, b_val, init_state, init_val, bk_Wq, bk_Wk, kb_Wq, kb_Wk,
           pp_Wq, pp_Wk, kv_g, kv_b, br_g, br_b, pn_g, pn_b):
    B = b_state.shape[0]

    k_state1 = _kstate(init_state)
    kval = _xln(jnp.broadcast_to(init_val[None], (B, N, DIM)), kv_g, kv_b)
    nk = _xln(kval, kv_g, kv_b)

    q1, k1, dirs1, gate1 = _prep(b_val, nk, b_state, bk_Wq, bk_Wk)
    d_state1, d_val1 = _route(k1, q1, gate1, dirs1)
    routed_state = k_state1 + d_state1
    routed_val = _xln(kval + d_val1, kv_g, kv_b)

    nv = _xln(routed_val, pn_g, pn_b)
    q2, k2, dirs2, gate2 = _prep(nv, nv, routed_state, pp_Wq, pp_Wk)
    d_state2, d_val2 = _route(k2, q2, gate2, dirs2)
    prop_state = routed_state + d_state2
    prop_val = _xln(routed_val + d_val2, kv_g, kv_b)

    nk2 = _xln(prop_val, kv_g, kv_b)
    nb = _xln(b_val, br_g, br_b)
    q3, k3, dirs3, gate3 = _prep(nk2, nb, prop_state, kb_Wq, kb_Wk)
    bd_state, bd_val = _route(k3, q3, gate3, dirs3)

    return (routed_state, routed_val, prop_state, prop_val, bd_state, bd_val)
